# Initial kernel scaffold; baseline (speedup 1.0000x reference)
import functools

import jax
import jax.numpy as jnp
from jax import lax
from jax.experimental import pallas as pl
from jax.experimental.pallas import tpu as pltpu
from jax.experimental.pallas import tpu_sc as plsc

_N = 10000
_E = 320000
_D = 128

_NC = 2
_NS = 16
_NW = _NC * _NS
_EPW = _E // _NW
_CH = 80
_NCH = _EPW // _CH
_WCH = 1000
_WT = _N // _WCH


def _make_sc_agg(with_cnt):
    mesh = plsc.VectorSubcoreMesh(core_axis_name="c", subcore_axis_name="s")
    out_type = [jax.ShapeDtypeStruct((_NC * _N, _D), jnp.float32)]
    scratch_types = [
        pltpu.VMEM((_CH,), jnp.int32),
        pltpu.VMEM((_CH,), jnp.int32),
        pltpu.VMEM((_CH, _D), jnp.float32),
        pltpu.VMEM_SHARED((_N, _D), jnp.float32),
        pltpu.SemaphoreType.DMA,
    ]
    if with_cnt:
        out_type.append(jax.ShapeDtypeStruct((_NC * _N,), jnp.float32))
        scratch_types += [
            pltpu.VMEM((_CH,), jnp.float32),
            pltpu.VMEM_SHARED((_N,), jnp.float32),
        ]

    def body(y_hbm, src_hbm, dst_hbm, zrow_hbm, zcnt_hbm, *rest):
        if with_cnt:
            (out_hbm, cnt_hbm, src_v, dst_v, rows_v, agg_sh, sem,
             ones_v, cnt_sh) = rest
        else:
            out_hbm, src_v, dst_v, rows_v, agg_sh, sem = rest
        cid = lax.axis_index("c")
        sid = lax.axis_index("s")
        wid = sid * _NC + cid

        @pl.when(sid < _WT)
        def _():
            pltpu.sync_copy(zrow_hbm, agg_sh.at[pl.ds(sid * _WCH, _WCH)])
            if with_cnt:
                pltpu.sync_copy(zcnt_hbm, cnt_sh.at[pl.ds(sid * _WCH, _WCH)])

        if with_cnt:
            for i in range(_CH // 16):
                ones_v[pl.ds(i * 16, 16)] = jnp.full((16,), 1.0, jnp.float32)
        plsc.subcore_barrier()

        ebase = wid * _EPW

        def step(k, carry):
            off = ebase + k * _CH
            pltpu.sync_copy(src_hbm.at[pl.ds(off, _CH)], src_v)
            pltpu.sync_copy(dst_hbm.at[pl.ds(off, _CH)], dst_v)
            pltpu.async_copy(y_hbm.at[src_v], rows_v, sem).wait()
            pltpu.sync_copy(rows_v, agg_sh.at[dst_v], add=True)
            if with_cnt:
                pltpu.sync_copy(ones_v, cnt_sh.at[dst_v], add=True)
            return carry

        lax.fori_loop(0, _NCH, step, 0)
        plsc.subcore_barrier()

        @pl.when(sid < _WT)
        def _():
            o = cid * _N + sid * _WCH
            pltpu.sync_copy(agg_sh.at[pl.ds(sid * _WCH, _WCH)],
                            out_hbm.at[pl.ds(o, _WCH)])
            if with_cnt:
                pltpu.sync_copy(cnt_sh.at[pl.ds(sid * _WCH, _WCH)],
                                cnt_hbm.at[pl.ds(o, _WCH)])

    return pl.kernel(body, mesh=mesh, out_type=tuple(out_type),
                     scratch_types=tuple(scratch_types))


_BR = 1000


def _tc1_body(x_ref, wl_ref, wr_ref, bl_ref, y_ref, z_ref):
    xb = x_ref[...]
    y_ref[...] = jnp.dot(xb, wl_ref[...], preferred_element_type=jnp.float32)
    z_ref[...] = (jnp.dot(xb, wr_ref[...], preferred_element_type=jnp.float32)
                  + bl_ref[...])


def _tc2_body(sa_ref, sb_ref, ca_ref, cb_ref, z1_ref, wl_ref, wr_ref, bl_ref,
              y_ref, z_ref):
    r = 1.0 / jnp.maximum(ca_ref[...] + cb_ref[...], 1.0)
    h = jnp.maximum((sa_ref[...] + sb_ref[...]) * r + z1_ref[...], 0.0)
    y_ref[...] = jnp.dot(h, wl_ref[...], preferred_element_type=jnp.float32)
    z_ref[...] = (jnp.dot(h, wr_ref[...], preferred_element_type=jnp.float32)
                  + bl_ref[...])


def _tc3_body(sa_ref, sb_ref, ca_ref, cb_ref, z2_ref, o_ref):
    r = 1.0 / jnp.maximum(ca_ref[...] + cb_ref[...], 1.0)
    o_ref[...] = (sa_ref[...] + sb_ref[...]) * r + z2_ref[...]


_row_spec = pl.BlockSpec((_BR, _D), lambda i: (i, 0))
_cnt_spec = pl.BlockSpec((_BR, 1), lambda i: (i, 0))
_w_spec = pl.BlockSpec((_D, _D), lambda i: (0, 0))
_b_spec = pl.BlockSpec((1, _D), lambda i: (0, 0))
_row_out = jax.ShapeDtypeStruct((_N, _D), jnp.float32)

_tc1 = pl.pallas_call(
    _tc1_body, grid=(_N // _BR,),
    in_specs=[_row_spec, _w_spec, _w_spec, _b_spec],
    out_specs=[_row_spec, _row_spec],
    out_shape=[_row_out, _row_out],
)

_tc2 = pl.pallas_call(
    _tc2_body, grid=(_N // _BR,),
    in_specs=[_row_spec, _row_spec, _cnt_spec, _cnt_spec, _row_spec,
              _w_spec, _w_spec, _b_spec],
    out_specs=[_row_spec, _row_spec],
    out_shape=[_row_out, _row_out],
)

_tc3 = pl.pallas_call(
    _tc3_body, grid=(_N // _BR,),
    in_specs=[_row_spec, _row_spec, _cnt_spec, _cnt_spec, _row_spec],
    out_specs=_row_spec,
    out_shape=_row_out,
)

_sc_agg_cnt = _make_sc_agg(True)
_sc_agg = _make_sc_agg(False)


def kernel(x, edge_index, Wl1, bl1, Wr1, Wl2, bl2, Wr2):
    src = edge_index[0].astype(jnp.int32)
    dst = edge_index[1].astype(jnp.int32)
    zrow = jnp.zeros((_WCH, _D), jnp.float32)
    zcnt = jnp.zeros((_WCH,), jnp.float32)

    y1, z1 = _tc1(x, Wl1.T, Wr1.T, bl1.reshape(1, _D))
    s1, cnt = _sc_agg_cnt(y1, src, dst, zrow, zcnt)
    ca = cnt[:_N].reshape(_N, 1)
    cb = cnt[_N:].reshape(_N, 1)
    y2, z2 = _tc2(s1[:_N], s1[_N:], ca, cb, z1,
                  Wl2.T, Wr2.T, bl2.reshape(1, _D))
    (s2,) = _sc_agg(y2, src, dst, zrow, zcnt)
    out = _tc3(s2[:_N], s2[_N:], ca, cb, z2)
    return out


# speedup vs baseline: 5.0366x; 5.0366x over previous
import functools

import jax
import jax.numpy as jnp
from jax import lax
from jax.experimental import pallas as pl
from jax.experimental.pallas import tpu as pltpu
from jax.experimental.pallas import tpu_sc as plsc

_N = 10000
_E = 320000
_D = 128
_DA = 144

_NC = 2
_NS = 16
_NW = _NC * _NS
_EPW = _E // _NW
_CH = 80
_NCH = _EPW // _CH
_WCH = 1000
_WT = _N // _WCH


def _make_sc_agg(width):
    mesh = plsc.VectorSubcoreMesh(core_axis_name="c", subcore_axis_name="s")
    out_type = jax.ShapeDtypeStruct((_NC * _N, width), jnp.float32)
    scratch_types = (
        pltpu.VMEM((_CH,), jnp.int32),
        pltpu.VMEM((_CH,), jnp.int32),
        pltpu.VMEM((_CH, width), jnp.float32),
        pltpu.VMEM_SHARED((_N, width), jnp.float32),
        pltpu.SemaphoreType.DMA,
    )

    def body(y_hbm, src_hbm, dst_hbm, zrow_hbm, out_hbm,
             src_v, dst_v, rows_v, agg_sh, sem):
        cid = lax.axis_index("c")
        sid = lax.axis_index("s")
        wid = sid * _NC + cid

        @pl.when(sid < _WT)
        def _():
            pltpu.sync_copy(zrow_hbm, agg_sh.at[pl.ds(sid * _WCH, _WCH)])

        plsc.subcore_barrier()

        ebase = wid * _EPW

        def step(k, carry):
            off = ebase + k * _CH
            pltpu.sync_copy(src_hbm.at[pl.ds(off, _CH)], src_v)
            pltpu.sync_copy(dst_hbm.at[pl.ds(off, _CH)], dst_v)
            pltpu.async_copy(y_hbm.at[src_v], rows_v, sem).wait()
            pltpu.sync_copy(rows_v, agg_sh.at[dst_v], add=True)
            return carry

        lax.fori_loop(0, _NCH, step, 0)
        plsc.subcore_barrier()

        @pl.when(sid < _WT)
        def _():
            o = cid * _N + sid * _WCH
            pltpu.sync_copy(agg_sh.at[pl.ds(sid * _WCH, _WCH)],
                            out_hbm.at[pl.ds(o, _WCH)])

    return pl.kernel(
        body, mesh=mesh, out_type=out_type, scratch_types=scratch_types,
        compiler_params=pltpu.CompilerParams(use_tc_tiling_on_sc=False))


_BR = 1000


def _tc1_body(x_ref, wl_ref, wr_ref, bl_ref, y_ref, z_ref):
    xb = x_ref[...]
    xw = jnp.dot(xb, wl_ref[...], preferred_element_type=jnp.float32)
    pad = jnp.where(
        lax.broadcasted_iota(jnp.int32, (_BR, _DA - _D), 1) == 0, 1.0, 0.0)
    y_ref[...] = jnp.concatenate([xw, pad], axis=1)
    z_ref[...] = (jnp.dot(xb, wr_ref[...], preferred_element_type=jnp.float32)
                  + bl_ref[...])


def _tc2_body(sa_ref, sb_ref, z1_ref, wl_ref, wr_ref, bl_ref, y_ref, z_ref):
    s = sa_ref[...] + sb_ref[...]
    r = 1.0 / jnp.maximum(s[:, _D:_D + 1], 1.0)
    h = jnp.maximum(s[:, :_D] * r + z1_ref[...], 0.0)
    y_ref[...] = jnp.dot(h, wl_ref[...], preferred_element_type=jnp.float32)
    z_ref[...] = (jnp.dot(h, wr_ref[...], preferred_element_type=jnp.float32)
                  + bl_ref[...])


def _tc3_body(sa_ref, sb_ref, ca_ref, cb_ref, z2_ref, o_ref):
    r = 1.0 / jnp.maximum(ca_ref[...] + cb_ref[...], 1.0)
    o_ref[...] = (sa_ref[...] + sb_ref[...]) * r + z2_ref[...]


_row_spec = pl.BlockSpec((_BR, _D), lambda i: (i, 0))
_aug_spec = pl.BlockSpec((_BR, _DA), lambda i: (i, 0))
_cnt_spec = pl.BlockSpec((_BR, 1), lambda i: (i, 0))
_w_spec = pl.BlockSpec((_D, _D), lambda i: (0, 0))
_b_spec = pl.BlockSpec((1, _D), lambda i: (0, 0))
_row_out = jax.ShapeDtypeStruct((_N, _D), jnp.float32)
_aug_out = jax.ShapeDtypeStruct((_N, _DA), jnp.float32)

_tc1 = pl.pallas_call(
    _tc1_body, grid=(_N // _BR,),
    in_specs=[_row_spec, _w_spec, _w_spec, _b_spec],
    out_specs=[_aug_spec, _row_spec],
    out_shape=[_aug_out, _row_out],
)

_tc2 = pl.pallas_call(
    _tc2_body, grid=(_N // _BR,),
    in_specs=[_aug_spec, _aug_spec, _row_spec, _w_spec, _w_spec, _b_spec],
    out_specs=[_row_spec, _row_spec],
    out_shape=[_row_out, _row_out],
)

_tc3 = pl.pallas_call(
    _tc3_body, grid=(_N // _BR,),
    in_specs=[_row_spec, _row_spec, _cnt_spec, _cnt_spec, _row_spec],
    out_specs=_row_spec,
    out_shape=_row_out,
)

_sc_agg_aug = _make_sc_agg(_DA)
_sc_agg = _make_sc_agg(_D)


def kernel(x, edge_index, Wl1, bl1, Wr1, Wl2, bl2, Wr2):
    src = edge_index[0].astype(jnp.int32)
    dst = edge_index[1].astype(jnp.int32)
    zaug = jnp.zeros((_WCH, _DA), jnp.float32)
    zrow = jnp.zeros((_WCH, _D), jnp.float32)

    y1, z1 = _tc1(x, Wl1.T, Wr1.T, bl1.reshape(1, _D))
    s1 = _sc_agg_aug(y1, src, dst, zaug)
    s1a, s1b = s1[:_N], s1[_N:]
    y2, z2 = _tc2(s1a, s1b, z1, Wl2.T, Wr2.T, bl2.reshape(1, _D))
    s2 = _sc_agg(y2, src, dst, zrow)
    ca = lax.slice(s1a, (0, _D), (_N, _D + 1))
    cb = lax.slice(s1b, (0, _D), (_N, _D + 1))
    out = _tc3(s2[:_N], s2[_N:], ca, cb, z2)
    return out


# speedup vs baseline: 11.3750x; 2.2585x over previous
import functools

import jax
import jax.numpy as jnp
from jax import lax
from jax.experimental import pallas as pl
from jax.experimental.pallas import tpu as pltpu
from jax.experimental.pallas import tpu_sc as plsc

_N = 10000
_E = 320000
_D = 128
_DC = 16

_NC = 2
_NS = 16
_NW = _NC * _NS
_EPW = _E // _NW
_CH = 80
_NCH = _EPW // _CH
_WCH = 1000
_WT = _N // _WCH

_mesh = plsc.VectorSubcoreMesh(core_axis_name="c", subcore_axis_name="s")
_sc_params = pltpu.CompilerParams(use_tc_tiling_on_sc=False)


def _agg_body(y_hbm, src_hbm, dst_hbm, zrow_hbm, out_hbm,
              src_v, dst_v, buf0, buf1, agg_sh, sem0, sem1):
    cid = lax.axis_index("c")
    sid = lax.axis_index("s")
    wid = sid * _NC + cid

    @pl.when(sid < _WT)
    def _():
        pltpu.sync_copy(zrow_hbm, agg_sh.at[pl.ds(sid * _WCH, _WCH)])

    pltpu.sync_copy(src_hbm.at[pl.ds(wid * _EPW, _EPW)], src_v)
    pltpu.sync_copy(dst_hbm.at[pl.ds(wid * _EPW, _EPW)], dst_v)
    plsc.subcore_barrier()

    def sidx(k):
        return src_v.at[pl.ds(k * _CH, _CH)]

    def didx(k):
        return dst_v.at[pl.ds(k * _CH, _CH)]

    pltpu.async_copy(y_hbm.at[sidx(0)], buf0, sem0)

    def step(g, carry):
        k0 = 2 * g
        pltpu.async_copy(y_hbm.at[sidx(k0 + 1)], buf1, sem1)
        pltpu.make_async_copy(y_hbm.at[sidx(k0)], buf0, sem0).wait()
        pltpu.sync_copy(buf0, agg_sh.at[didx(k0)], add=True)
        pltpu.async_copy(y_hbm.at[sidx(k0 + 2)], buf0, sem0)
        pltpu.make_async_copy(y_hbm.at[sidx(k0 + 1)], buf1, sem1).wait()
        pltpu.sync_copy(buf1, agg_sh.at[didx(k0 + 1)], add=True)
        return carry

    lax.fori_loop(0, (_NCH - 1) // 2, step, 0)
    pltpu.make_async_copy(y_hbm.at[sidx(_NCH - 1)], buf0, sem0).wait()
    pltpu.sync_copy(buf0, agg_sh.at[didx(_NCH - 1)], add=True)
    plsc.subcore_barrier()

    @pl.when(sid < _WT)
    def _():
        o = cid * _N + sid * _WCH
        pltpu.sync_copy(agg_sh.at[pl.ds(sid * _WCH, _WCH)],
                        out_hbm.at[pl.ds(o, _WCH)])


_sc_agg = pl.kernel(
    _agg_body, mesh=_mesh,
    out_type=jax.ShapeDtypeStruct((_NC * _N, _D), jnp.float32),
    scratch_types=(
        pltpu.VMEM((_EPW,), jnp.int32),
        pltpu.VMEM((_EPW,), jnp.int32),
        pltpu.VMEM((_CH, _D), jnp.float32),
        pltpu.VMEM((_CH, _D), jnp.float32),
        pltpu.VMEM_SHARED((_N, _D), jnp.float32),
        pltpu.SemaphoreType.DMA,
        pltpu.SemaphoreType.DMA,
    ),
    compiler_params=_sc_params)


def _cnt_body(dst_hbm, zcnt_hbm, out_hbm, dst_v, ones_v, cnt_sh):
    cid = lax.axis_index("c")
    sid = lax.axis_index("s")
    wid = sid * _NC + cid

    @pl.when(sid < _WT)
    def _():
        pltpu.sync_copy(zcnt_hbm, cnt_sh.at[pl.ds(sid * _WCH, _WCH)])

    pltpu.sync_copy(dst_hbm.at[pl.ds(wid * _EPW, _EPW)], dst_v)
    one_row = jnp.full((_DC,), 1.0, jnp.float32)
    def fill(i, carry):
        ones_v[i, :] = one_row
        return carry
    lax.fori_loop(0, _CH, fill, 0)
    plsc.subcore_barrier()

    def step(k, carry):
        pltpu.sync_copy(ones_v, cnt_sh.at[dst_v.at[pl.ds(k * _CH, _CH)]],
                        add=True)
        return carry

    lax.fori_loop(0, _NCH, step, 0)
    plsc.subcore_barrier()

    @pl.when(sid < _WT)
    def _():
        o = cid * _N + sid * _WCH
        pltpu.sync_copy(cnt_sh.at[pl.ds(sid * _WCH, _WCH)],
                        out_hbm.at[pl.ds(o, _WCH)])


_sc_cnt = pl.kernel(
    _cnt_body, mesh=_mesh,
    out_type=jax.ShapeDtypeStruct((_NC * _N, _DC), jnp.float32),
    scratch_types=(
        pltpu.VMEM((_EPW,), jnp.int32),
        pltpu.VMEM((_CH, _DC), jnp.float32),
        pltpu.VMEM_SHARED((_N, _DC), jnp.float32),
    ),
    compiler_params=_sc_params)


_BR = 1000


def _tc1_body(x_ref, wl_ref, wr_ref, bl_ref, y_ref, z_ref):
    xb = x_ref[...]
    y_ref[...] = jnp.dot(xb, wl_ref[...], preferred_element_type=jnp.float32)
    z_ref[...] = (jnp.dot(xb, wr_ref[...], preferred_element_type=jnp.float32)
                  + bl_ref[...])


def _recip(ca_ref, cb_ref):
    cnt = ca_ref[...] + cb_ref[...]
    return 1.0 / jnp.maximum(cnt[:, :1], 1.0)


def _tc2_body(sa_ref, sb_ref, ca_ref, cb_ref, z1_ref, wl_ref, wr_ref, bl_ref,
              y_ref, z_ref):
    r = _recip(ca_ref, cb_ref)
    h = jnp.maximum((sa_ref[...] + sb_ref[...]) * r + z1_ref[...], 0.0)
    y_ref[...] = jnp.dot(h, wl_ref[...], preferred_element_type=jnp.float32)
    z_ref[...] = (jnp.dot(h, wr_ref[...], preferred_element_type=jnp.float32)
                  + bl_ref[...])


def _tc3_body(sa_ref, sb_ref, ca_ref, cb_ref, z2_ref, o_ref):
    r = _recip(ca_ref, cb_ref)
    o_ref[...] = (sa_ref[...] + sb_ref[...]) * r + z2_ref[...]


_row_spec = pl.BlockSpec((_BR, _D), lambda i: (i, 0))
_cnt_spec = pl.BlockSpec((_BR, _DC), lambda i: (i, 0))
_w_spec = pl.BlockSpec((_D, _D), lambda i: (0, 0))
_b_spec = pl.BlockSpec((1, _D), lambda i: (0, 0))
_row_out = jax.ShapeDtypeStruct((_N, _D), jnp.float32)

_tc1 = pl.pallas_call(
    _tc1_body, grid=(_N // _BR,),
    in_specs=[_row_spec, _w_spec, _w_spec, _b_spec],
    out_specs=[_row_spec, _row_spec],
    out_shape=[_row_out, _row_out],
)

_tc2 = pl.pallas_call(
    _tc2_body, grid=(_N // _BR,),
    in_specs=[_row_spec, _row_spec, _cnt_spec, _cnt_spec, _row_spec,
              _w_spec, _w_spec, _b_spec],
    out_specs=[_row_spec, _row_spec],
    out_shape=[_row_out, _row_out],
)

_tc3 = pl.pallas_call(
    _tc3_body, grid=(_N // _BR,),
    in_specs=[_row_spec, _row_spec, _cnt_spec, _cnt_spec, _row_spec],
    out_specs=_row_spec,
    out_shape=_row_out,
)


def kernel(x, edge_index, Wl1, bl1, Wr1, Wl2, bl2, Wr2):
    src = edge_index[0].astype(jnp.int32)
    dst = edge_index[1].astype(jnp.int32)
    zrow = jnp.zeros((_WCH, _D), jnp.float32)
    zcnt = jnp.zeros((_WCH, _DC), jnp.float32)

    cnt = _sc_cnt(dst, zcnt)
    ca, cb = cnt[:_N], cnt[_N:]
    y1, z1 = _tc1(x, Wl1.T, Wr1.T, bl1.reshape(1, _D))
    s1 = _sc_agg(y1, src, dst, zrow)
    y2, z2 = _tc2(s1[:_N], s1[_N:], ca, cb, z1,
                  Wl2.T, Wr2.T, bl2.reshape(1, _D))
    s2 = _sc_agg(y2, src, dst, zrow)
    out = _tc3(s2[:_N], s2[_N:], ca, cb, z2)
    return out
